# baseline (speedup 1.0000x reference)
import functools

import jax
import jax.numpy as jnp
from jax import lax
from jax.experimental import pallas as pl
from jax.experimental.pallas import tpu as pltpu
from jax.experimental.pallas import tpu_sc as plsc

_B = 1024
_S = 200
_D = 128
_N = _B * _S
_NW = 32
_BLK = 128
_ROWS = _N // 128
_RPW = _ROWS // _NW
_NBLK = _RPW
_NBUF = 2
_G = 16
_NG = _BLK // _G


def _lane_sum(v):
    lanes = lax.iota(jnp.int32, 16)
    for k in (1, 2, 4, 8):
        idx = lax.bitwise_xor(lanes, jnp.int32(k))
        v = v + v.at[idx].get(mode="promise_in_bounds")
    return v


def _ln_one(bufA, obuf, i, row, posbuf2, gbv):
    e = []
    for k in range(8):
        e.append(bufA[i, pl.ds(16 * k, 16)] + posbuf2[row, pl.ds(16 * k, 16)])
    s01 = e[0] + e[1]
    s23 = e[2] + e[3]
    s45 = e[4] + e[5]
    s67 = e[6] + e[7]
    svec = (s01 + s23) + (s45 + s67)
    q01 = e[0] * e[0] + e[1] * e[1]
    q23 = e[2] * e[2] + e[3] * e[3]
    q45 = e[4] * e[4] + e[5] * e[5]
    q67 = e[6] * e[6] + e[7] * e[7]
    qvec = (q01 + q23) + (q45 + q67)
    mean = _lane_sum(svec) * jnp.float32(1.0 / 128.0)
    ex2 = _lane_sum(qvec) * jnp.float32(1.0 / 128.0)
    x = ex2 - mean * mean + jnp.float32(1e-6)
    xi = lax.bitcast_convert_type(x, jnp.int32)
    yi = jnp.int32(0x5F3759DF) - lax.shift_right_arithmetic(xi, jnp.int32(1))
    y = lax.bitcast_convert_type(yi, jnp.float32)
    half_x = jnp.float32(0.5) * x
    for _ in range(2):
        y = y * (jnp.float32(1.5) - half_x * y * y)
    for k in range(8):
        obuf[i, pl.ds(16 * k, 16)] = (e[k] - mean) * y * gbv[k] + gbv[8 + k]


def _ln_block(bufA, obuf, tt_v, b, s_start, posbuf2, gb_v):
    gbs = tuple(gb_v[0, pl.ds(16 * k, 16)] for k in range(8)) + tuple(
        gb_v[1, pl.ds(16 * k, 16)] for k in range(8))

    @plsc.parallel_loop(0, _NG, step=1, carry=(s_start,) + gbs)
    def grp(g, carry):
        s0 = carry[0]
        gbv = carry[1:]
        base16 = pl.multiple_of(g * _G, 16)
        tvec = tt_v[b, pl.ds(base16, 16)] * jnp.int32(_S)
        for j in range(_G):
            sj = s0 + jnp.int32(j)
            sj = jnp.where(sj >= jnp.int32(_S), sj - jnp.int32(_S), sj)
            row = tvec[j] + sj
            _ln_one(bufA, obuf, base16 + jnp.int32(j), row, posbuf2, gbv)
        s0n = s0 + jnp.int32(_G)
        s0n = jnp.where(s0n >= jnp.int32(_S), s0n - jnp.int32(_S), s0n)
        return (s0n,) + gbv


def _sc_kernel(ids_hbm, tt_hbm, word_hbm, pos_hbm, gbt_hbm, out_hbm,
               itt_v, bufA0, bufA1, obuf0, obuf1, posbuf2, gbt_v,
               semA0, semA1, semO0, semO1):
    c = lax.axis_index("c")
    s = lax.axis_index("s")
    wid = s * 2 + c
    idx_v = itt_v.at[pl.ds(0, _RPW)]
    tt_v = itt_v.at[pl.ds(_RPW, _RPW)]
    pltpu.sync_copy(ids_hbm.at[wid], idx_v)
    pltpu.sync_copy(tt_hbm.at[wid], tt_v)
    pltpu.sync_copy(gbt_hbm, gbt_v)
    pltpu.sync_copy(pos_hbm.at[pl.ds(0, _S)], posbuf2.at[pl.ds(0, _S)])
    pltpu.sync_copy(pos_hbm.at[pl.ds(0, _S)], posbuf2.at[pl.ds(_S, _S)])

    ty0 = tuple(gbt_v[2, pl.ds(16 * k, 16)] for k in range(8))
    ty1 = tuple(gbt_v[3, pl.ds(16 * k, 16)] for k in range(8))

    @plsc.parallel_loop(0, _S, step=1, unroll=2, carry=ty0 + ty1)
    def _shift(r, t01):
        for k in range(8):
            posbuf2[r, pl.ds(16 * k, 16)] = (
                posbuf2[r, pl.ds(16 * k, 16)] + t01[k])
        r2 = r + jnp.int32(_S)
        for k in range(8):
            posbuf2[r2, pl.ds(16 * k, 16)] = (
                posbuf2[r2, pl.ds(16 * k, 16)] + t01[8 + k])
        return t01

    bufA = (bufA0, bufA1)
    obuf = (obuf0, obuf1)
    semA = (semA0, semA1)
    semO = (semO0, semO1)

    def gatherA(t, p):
        return pltpu.make_async_copy(word_hbm.at[idx_v.at[t]], bufA[p], semA[p])

    def ocopy(t, p):
        base = pl.multiple_of(wid * (_RPW * 128) + t * _BLK, _BLK)
        return pltpu.make_async_copy(obuf[p], out_hbm.at[pl.ds(base, _BLK)], semO[p])

    for p in range(_NBUF):
        gatherA(p, p).start()

    def pair(g, s_start):
        for p in range(_NBUF):
            t = g * _NBUF + p
            gatherA(t, p).wait()

            @pl.when(t >= _NBUF)
            def _():
                ocopy(t - _NBUF, p).wait()

            _ln_block(bufA[p], obuf[p], tt_v, t, s_start, posbuf2, gbt_v)
            ocopy(t, p).start()

            @pl.when(t + _NBUF < _NBLK)
            def _():
                gatherA(t + _NBUF, p).start()

            s_start = s_start + jnp.int32(_BLK - _S)
            s_start = jnp.where(s_start < 0, s_start + jnp.int32(_S), s_start)
        return s_start

    lax.fori_loop(0, _NBLK // _NBUF, pair, jnp.int32(0), unroll=False)

    for p in range(_NBUF):
        ocopy(_NBLK - _NBUF + p, p).wait()


@functools.partial(jax.jit, static_argnums=())
def _run(ids2d, tt2d, word_table, pos_table, gbt):
    mesh = plsc.VectorSubcoreMesh(core_axis_name="c", subcore_axis_name="s")
    f = pl.kernel(
        _sc_kernel,
        mesh=mesh,
        out_type=jax.ShapeDtypeStruct((_N, _D), jnp.float32),
        scratch_types=[
            pltpu.VMEM((2 * _RPW, 128), jnp.int32),
            pltpu.VMEM((_BLK, _D), jnp.float32),
            pltpu.VMEM((_BLK, _D), jnp.float32),
            pltpu.VMEM((_BLK, _D), jnp.float32),
            pltpu.VMEM((_BLK, _D), jnp.float32),
            pltpu.VMEM((2 * _S, _D), jnp.float32),
            pltpu.VMEM((4, _D), jnp.float32),
            pltpu.SemaphoreType.DMA,
            pltpu.SemaphoreType.DMA,
            pltpu.SemaphoreType.DMA,
            pltpu.SemaphoreType.DMA,
        ],
    )
    return f(ids2d, tt2d, word_table, pos_table, gbt)


def kernel(input_ids, token_type_ids, word_table, pos_table, type_table, gamma, beta):
    ids2d = input_ids.astype(jnp.int32).reshape(_NW, _RPW, 128)
    tt2d = token_type_ids.astype(jnp.int32).reshape(_NW, _RPW, 128)
    gbt = jnp.concatenate([jnp.stack([gamma, beta], axis=0), type_table], axis=0)
    out = _run(ids2d, tt2d, word_table, pos_table, gbt)
    return out.reshape(_B, _S, _D)


# speedup vs baseline: 1.6596x; 1.6596x over previous
import functools

import jax
import jax.numpy as jnp
from jax import lax
from jax.experimental import pallas as pl
from jax.experimental.pallas import tpu as pltpu
from jax.experimental.pallas import tpu_sc as plsc

_B = 1024
_S = 200
_D = 128
_N = _B * _S
_NW = 32
_BLK = 128
_ROWS = _N // 128
_RPW = _ROWS // _NW
_NBLK = _RPW
_NBUF = 2


def _lane_sum(v):
    lanes = lax.iota(jnp.int32, 16)
    for k in (1, 2, 4, 8):
        idx = lax.bitwise_xor(lanes, jnp.int32(k))
        v = v + v.at[idx].get(mode="promise_in_bounds")
    return v


def _ln_block(bufA, tt_v, b, tok0, posbuf, obuf, gb_v, ty_v):
    gbs = tuple(gb_v[0, pl.ds(16 * k, 16)] for k in range(8)) + tuple(
        gb_v[1, pl.ds(16 * k, 16)] for k in range(8))

    @plsc.parallel_loop(0, _BLK, step=1, unroll=2, carry=gbs)
    def tok(i, gb):
        s_pos = lax.rem(tok0 + i, jnp.int32(_S))
        base16 = pl.multiple_of(lax.bitwise_and(i, jnp.int32(-16)), 16)
        lane = lax.bitwise_and(i, jnp.int32(15))
        tvec = tt_v[b, pl.ds(base16, 16)]
        t_b = tvec.at[jnp.broadcast_to(lane, (16,))].get(mode="promise_in_bounds")
        t_f = t_b.astype(jnp.float32)
        e = []
        for k in range(8):
            e.append(bufA[i, pl.ds(16 * k, 16)]
                     + (posbuf[s_pos, pl.ds(16 * k, 16)]
                        + t_f * ty_v[1, pl.ds(16 * k, 16)]))
        s01 = e[0] + e[1]
        s23 = e[2] + e[3]
        s45 = e[4] + e[5]
        s67 = e[6] + e[7]
        svec = (s01 + s23) + (s45 + s67)
        q01 = e[0] * e[0] + e[1] * e[1]
        q23 = e[2] * e[2] + e[3] * e[3]
        q45 = e[4] * e[4] + e[5] * e[5]
        q67 = e[6] * e[6] + e[7] * e[7]
        qvec = (q01 + q23) + (q45 + q67)
        mean = _lane_sum(svec) * jnp.float32(1.0 / 128.0)
        ex2 = _lane_sum(qvec) * jnp.float32(1.0 / 128.0)
        x = ex2 - mean * mean + jnp.float32(1e-6)
        xi = lax.bitcast_convert_type(x, jnp.int32)
        yi = jnp.int32(0x5F3759DF) - lax.shift_right_arithmetic(xi, jnp.int32(1))
        y = lax.bitcast_convert_type(yi, jnp.float32)
        half_x = jnp.float32(0.5) * x
        for _ in range(2):
            y = y * (jnp.float32(1.5) - half_x * y * y)
        for k in range(8):
            obuf[i, pl.ds(16 * k, 16)] = (e[k] - mean) * y * gb[k] + gb[8 + k]
        return gb


def _sc_kernel(ids_hbm, tt_hbm, word_hbm, pos_hbm, ty_hbm, gb_hbm, out_hbm,
               idx_v, tt_v, bufA0, bufA1, obuf0, obuf1, posbuf, gb_v, ty_v,
               semA0, semA1, semO0, semO1):
    c = lax.axis_index("c")
    s = lax.axis_index("s")
    wid = s * 2 + c
    pltpu.sync_copy(ids_hbm.at[wid], idx_v)
    pltpu.sync_copy(tt_hbm.at[wid], tt_v)
    pltpu.sync_copy(gb_hbm, gb_v)
    pltpu.sync_copy(ty_hbm, ty_v)
    pltpu.sync_copy(pos_hbm.at[pl.ds(0, _S)], posbuf)

    ty0 = tuple(ty_v[0, pl.ds(16 * k, 16)] for k in range(8))
    for k in range(8):
        ty_v[1, pl.ds(16 * k, 16)] = ty_v[1, pl.ds(16 * k, 16)] - ty0[k]

    @plsc.parallel_loop(0, _S, step=1, unroll=2, carry=ty0)
    def _shift(r, t0):
        for k in range(8):
            posbuf[r, pl.ds(16 * k, 16)] = posbuf[r, pl.ds(16 * k, 16)] + t0[k]
        return t0

    bufA = (bufA0, bufA1)
    obuf = (obuf0, obuf1)
    semA = (semA0, semA1)
    semO = (semO0, semO1)

    def gatherA(t, p):
        return pltpu.make_async_copy(word_hbm.at[idx_v.at[t]], bufA[p], semA[p])

    def ocopy(t, p):
        base = pl.multiple_of(wid * (_RPW * 128) + t * _BLK, _BLK)
        return pltpu.make_async_copy(obuf[p], out_hbm.at[pl.ds(base, _BLK)], semO[p])

    for p in range(_NBUF):
        gatherA(p, p).start()

    tok_base = wid * (_RPW * 128)

    def pair(g, carry):
        for p in range(_NBUF):
            t = g * _NBUF + p
            gatherA(t, p).wait()

            @pl.when(t >= _NBUF)
            def _():
                ocopy(t - _NBUF, p).wait()

            _ln_block(bufA[p], tt_v, t, tok_base + t * _BLK, posbuf,
                      obuf[p], gb_v, ty_v)
            ocopy(t, p).start()

            @pl.when(t + _NBUF < _NBLK)
            def _():
                gatherA(t + _NBUF, p).start()
        return carry

    lax.fori_loop(0, _NBLK // _NBUF, pair, 0, unroll=False)

    for p in range(_NBUF):
        ocopy(_NBLK - _NBUF + p, p).wait()


@functools.partial(jax.jit, static_argnums=())
def _run(ids2d, tt2d, word_table, pos_table, ty, gb):
    mesh = plsc.VectorSubcoreMesh(core_axis_name="c", subcore_axis_name="s")
    f = pl.kernel(
        _sc_kernel,
        mesh=mesh,
        out_type=jax.ShapeDtypeStruct((_N, _D), jnp.float32),
        scratch_types=[
            pltpu.VMEM((_RPW, 128), jnp.int32),
            pltpu.VMEM((_RPW, 128), jnp.int32),
            pltpu.VMEM((_BLK, _D), jnp.float32),
            pltpu.VMEM((_BLK, _D), jnp.float32),
            pltpu.VMEM((_BLK, _D), jnp.float32),
            pltpu.VMEM((_BLK, _D), jnp.float32),
            pltpu.VMEM((_S, _D), jnp.float32),
            pltpu.VMEM((2, _D), jnp.float32),
            pltpu.VMEM((2, _D), jnp.float32),
            pltpu.SemaphoreType.DMA,
            pltpu.SemaphoreType.DMA,
            pltpu.SemaphoreType.DMA,
            pltpu.SemaphoreType.DMA,
        ],
    )
    return f(ids2d, tt2d, word_table, pos_table, ty, gb)


def kernel(input_ids, token_type_ids, word_table, pos_table, type_table, gamma, beta):
    ids2d = input_ids.astype(jnp.int32).reshape(_NW, _RPW, 128)
    tt2d = token_type_ids.astype(jnp.int32).reshape(_NW, _RPW, 128)
    gb = jnp.stack([gamma, beta], axis=0)
    out = _run(ids2d, tt2d, word_table, pos_table, type_table, gb)
    return out.reshape(_B, _S, _D)


# speedup vs baseline: 1.8437x; 1.1109x over previous
import functools

import jax
import jax.numpy as jnp
from jax import lax
from jax.experimental import pallas as pl
from jax.experimental.pallas import tpu as pltpu
from jax.experimental.pallas import tpu_sc as plsc

_B = 1024
_S = 200
_D = 128
_N = _B * _S
_NW = 32
_BLK = 128
_ROWS = _N // 128
_RPW = _ROWS // _NW
_NBLK = _RPW
_NBUF = 2


def _lane_sum(v):
    lanes = lax.iota(jnp.int32, 16)
    for k in (1, 2, 4, 8):
        idx = lax.bitwise_xor(lanes, jnp.int32(k))
        v = v + v.at[idx].get(mode="promise_in_bounds")
    return v


def _ln_block(bufA, stt, tok0, posbuf, obuf, gb_v):
    gbs = tuple(gb_v[0, pl.ds(16 * k, 16)] for k in range(8)) + tuple(
        gb_v[1, pl.ds(16 * k, 16)] for k in range(8))

    @plsc.parallel_loop(0, _BLK, step=1, unroll=2, carry=gbs)
    def tok(i, gb):
        s_pos = lax.rem(tok0 + i, jnp.int32(_S))
        row = stt[i] * jnp.int32(_S) + s_pos
        e = []
        for k in range(8):
            e.append(bufA[i, pl.ds(16 * k, 16)] + posbuf[row, pl.ds(16 * k, 16)])
        s01 = e[0] + e[1]
        s23 = e[2] + e[3]
        s45 = e[4] + e[5]
        s67 = e[6] + e[7]
        svec = (s01 + s23) + (s45 + s67)
        q01 = e[0] * e[0] + e[1] * e[1]
        q23 = e[2] * e[2] + e[3] * e[3]
        q45 = e[4] * e[4] + e[5] * e[5]
        q67 = e[6] * e[6] + e[7] * e[7]
        qvec = (q01 + q23) + (q45 + q67)
        mean = _lane_sum(svec) * jnp.float32(1.0 / 128.0)
        ex2 = _lane_sum(qvec) * jnp.float32(1.0 / 128.0)
        x = ex2 - mean * mean + jnp.float32(1e-6)
        xi = lax.bitcast_convert_type(x, jnp.int32)
        yi = jnp.int32(0x5F3759DF) - lax.shift_right_arithmetic(xi, jnp.int32(1))
        y = lax.bitcast_convert_type(yi, jnp.float32)
        half_x = jnp.float32(0.5) * x
        for _ in range(2):
            y = y * (jnp.float32(1.5) - half_x * y * y)
        for k in range(8):
            obuf[i, pl.ds(16 * k, 16)] = (e[k] - mean) * y * gb[k] + gb[8 + k]
        return gb


def _sc_kernel(ids_hbm, tt_hbm, word_hbm, pos_hbm, gbt_hbm, out_hbm,
               idx_v, bufA0, bufA1, obuf0, obuf1, posbuf, gbt_v, ttrow,
               stt, semA0, semA1, semO0, semO1, semT0, semT1):
    c = lax.axis_index("c")
    s = lax.axis_index("s")
    wid = s * 2 + c
    gb_v = gbt_v.at[pl.ds(0, 2)]
    ty_v = gbt_v.at[pl.ds(2, 2)]
    pltpu.sync_copy(ids_hbm.at[wid], idx_v)
    pltpu.sync_copy(gbt_hbm, gbt_v)
    pltpu.sync_copy(pos_hbm.at[pl.ds(0, _S)], posbuf.at[pl.ds(0, _S)])
    pltpu.sync_copy(pos_hbm.at[pl.ds(0, _S)], posbuf.at[pl.ds(_S, _S)])

    ty01 = tuple(ty_v[0, pl.ds(16 * k, 16)] for k in range(8)) + tuple(
        ty_v[1, pl.ds(16 * k, 16)] for k in range(8))

    @plsc.parallel_loop(0, _S, step=1, unroll=2, carry=ty01)
    def _shift(r, t01):
        for k in range(8):
            posbuf[r, pl.ds(16 * k, 16)] = posbuf[r, pl.ds(16 * k, 16)] + t01[k]
        r2 = r + jnp.int32(_S)
        for k in range(8):
            posbuf[r2, pl.ds(16 * k, 16)] = (
                posbuf[r2, pl.ds(16 * k, 16)] + t01[8 + k])
        return t01

    bufA = (bufA0, bufA1)
    obuf = (obuf0, obuf1)
    semA = (semA0, semA1)
    semO = (semO0, semO1)

    semT = (semT0, semT1)

    def gatherA(t, p):
        return pltpu.make_async_copy(word_hbm.at[idx_v.at[t]], bufA[p], semA[p])

    def gatherT(t, p):
        return pltpu.make_async_copy(tt_hbm.at[wid].at[t], ttrow.at[p], semT[p])

    def ocopy(t, p):
        base = pl.multiple_of(wid * (_RPW * 128) + t * _BLK, _BLK)
        return pltpu.make_async_copy(obuf[p], out_hbm.at[pl.ds(base, _BLK)], semO[p])

    for p in range(_NBUF):
        gatherA(p, p).start()
        gatherT(p, p).start()

    tok_base = wid * (_RPW * 128)

    def pair(g, carry):
        for p in range(_NBUF):
            t = g * _NBUF + p
            gatherA(t, p).wait()
            gatherT(t, p).wait()

            @pl.when(t >= _NBUF)
            def _():
                ocopy(t - _NBUF, p).wait()

            for c8 in range(8):
                tv = ttrow[p, pl.ds(16 * c8, 16)]
                for j in range(16):
                    stt[16 * c8 + j] = tv[j]
            _ln_block(bufA[p], stt, tok_base + t * _BLK, posbuf,
                      obuf[p], gb_v)
            ocopy(t, p).start()

            @pl.when(t + _NBUF < _NBLK)
            def _():
                gatherA(t + _NBUF, p).start()
                gatherT(t + _NBUF, p).start()
        return carry

    lax.fori_loop(0, _NBLK // _NBUF, pair, 0, unroll=False)

    for p in range(_NBUF):
        ocopy(_NBLK - _NBUF + p, p).wait()


@functools.partial(jax.jit, static_argnums=())
def _run(ids2d, tt2d, word_table, pos_table, gbt):
    mesh = plsc.VectorSubcoreMesh(core_axis_name="c", subcore_axis_name="s")
    f = pl.kernel(
        _sc_kernel,
        mesh=mesh,
        out_type=jax.ShapeDtypeStruct((_N, _D), jnp.float32),
        scratch_types=[
            pltpu.VMEM((_RPW, 128), jnp.int32),
            pltpu.VMEM((_BLK, _D), jnp.float32),
            pltpu.VMEM((_BLK, _D), jnp.float32),
            pltpu.VMEM((_BLK, _D), jnp.float32),
            pltpu.VMEM((_BLK, _D), jnp.float32),
            pltpu.VMEM((2 * _S, _D), jnp.float32),
            pltpu.VMEM((4, _D), jnp.float32),
            pltpu.VMEM((2, 128), jnp.int32),
            pltpu.SMEM((_BLK,), jnp.int32),
            pltpu.SemaphoreType.DMA,
            pltpu.SemaphoreType.DMA,
            pltpu.SemaphoreType.DMA,
            pltpu.SemaphoreType.DMA,
            pltpu.SemaphoreType.DMA,
            pltpu.SemaphoreType.DMA,
        ],
    )
    return f(ids2d, tt2d, word_table, pos_table, gbt)


def kernel(input_ids, token_type_ids, word_table, pos_table, type_table, gamma, beta):
    ids2d = input_ids.astype(jnp.int32).reshape(_NW, _RPW, 128)
    tt2d = token_type_ids.astype(jnp.int32).reshape(_NW, _RPW, 128)
    gbt = jnp.concatenate([jnp.stack([gamma, beta], axis=0), type_table], axis=0)
    out = _run(ids2d, tt2d, word_table, pos_table, gbt)
    return out.reshape(_B, _S, _D)
